# baseline (speedup 1.0000x reference)
import functools

import jax
import jax.numpy as jnp
from jax import lax
from jax.experimental import pallas as pl
from jax.experimental.pallas import tpu as pltpu
from jax.experimental.pallas import tpu_sc as plsc

_R = 2
_SX = (1.0 / (2.0 * 0.05 ** 2)) ** 0.5
_SH = (1.0 / (2.0 * 0.1 ** 2)) ** 0.5
_PAD = 1e4

_B, _H, _W = 4, 192, 640
_WP = 656
_HP = _H + 2 * _R
_NW = 32
_RPW = _H * _B // _NW
_BLK = 6
_NBLK = _RPW // _BLK
_PW = _BLK + 2 * _R
_L = 16


def _sc_body(xy1_h, dgt_h, msk_h, hsv_h, xy1p_h, dpp_h, hsvp_h,
             psum_h, cnt_h,
             xyzp_s, hsvp_s, dpp_v, xyzg_s, hsvg_s, dgt_v, msk_v,
             psum_st, cnt_st, sem):
    wid = lax.axis_index("s") * 2 + lax.axis_index("c")
    b = wid // 8
    y0 = (wid % 8) * _RPW

    acc = jnp.zeros((_L,), jnp.float32)
    cnt = jnp.zeros((_L,), jnp.float32)

    for k in range(_NBLK):
        yg = y0 + k * _BLK
        cps = []
        pb = (b * 3) * (_HP * _WP)
        gb = (b * 3) * (_H * _W)
        pb1 = b * (_HP * _WP)
        gb1 = b * (_H * _W)
        for c in range(3):
            cps.append(pltpu.async_copy(
                xy1p_h.at[pl.ds(pb + c * (_HP * _WP) + yg * _WP, _PW * _WP)],
                xyzp_s.at[pl.ds(c * (_PW * _WP), _PW * _WP)], sem))
        cps.append(pltpu.async_copy(
            dpp_h.at[pl.ds(pb1 + yg * _WP, _PW * _WP)], dpp_v, sem))
        for c in range(3):
            cps.append(pltpu.async_copy(
                hsvp_h.at[pl.ds(pb + c * (_HP * _WP) + yg * _WP, _PW * _WP)],
                hsvp_s.at[pl.ds(c * (_PW * _WP), _PW * _WP)], sem))
        for c in range(3):
            cps.append(pltpu.async_copy(
                xy1_h.at[pl.ds(gb + c * (_H * _W) + yg * _W, _BLK * _W)],
                xyzg_s.at[pl.ds(c * (_BLK * _W), _BLK * _W)], sem))
        cps.append(pltpu.async_copy(
            dgt_h.at[pl.ds(gb1 + yg * _W, _BLK * _W)], dgt_v, sem))
        for c in range(3):
            cps.append(pltpu.async_copy(
                hsv_h.at[pl.ds(gb + c * (_H * _W) + yg * _W, _BLK * _W)],
                hsvg_s.at[pl.ds(c * (_BLK * _W), _BLK * _W)], sem))
        cps.append(pltpu.async_copy(
            msk_h.at[pl.ds(gb1 + yg * _W, _BLK * _W)], msk_v, sem))
        for cp in cps:
            cp.wait()

        def pre_pred(j, carry):
            off = j * _L
            d = dpp_v[pl.ds(off, _L)] * _SX
            for c in range(3):
                sl = pl.ds(c * (_PW * _WP) + off, _L)
                xyzp_s[sl] = xyzp_s[sl] * d
                hsvp_s[sl] = hsvp_s[sl] * _SH
            return carry
        lax.fori_loop(0, _PW * _WP // _L, pre_pred, 0)

        def pre_gt(j, carry):
            off = j * _L
            d = dgt_v[pl.ds(off, _L)] * _SX
            for c in range(3):
                sl = pl.ds(c * (_BLK * _W) + off, _L)
                xyzg_s[sl] = xyzg_s[sl] * d
                hsvg_s[sl] = hsvg_s[sl] * _SH
            return carry
        lax.fori_loop(0, _BLK * _W // _L, pre_gt, 0)

        nchunk = _W // _L

        def row_loop(ri, carry0):
            def chunk(xc, carry):
                a_acc, a_cnt = carry
                goff = ri * _W + xc * _L
                gsl = pl.ds(goff, _L)
                m = msk_v[gsl]
                gx = [xyzg_s[pl.ds(c * (_BLK * _W) + goff, _L)] for c in range(3)]
                gh = [hsvg_s[pl.ds(c * (_BLK * _W) + goff, _L)] for c in range(3)]
                pbase = ri * _WP + xc * _L
                accs = []
                for dy in range(2 * _R + 1):
                    a = None
                    for dx in range(2 * _R + 1):
                        nb = pbase + dy * _WP + dx
                        sq = []
                        for c in range(3):
                            d = gx[c] - xyzp_s[pl.ds(c * (_PW * _WP) + nb, _L)]
                            sq.append(d * d)
                        for c in range(3):
                            d = gh[c] - hsvp_s[pl.ds(c * (_PW * _WP) + nb, _L)]
                            sq.append(d * d)
                        t = ((sq[0] + sq[1]) + (sq[2] + sq[3])) + (sq[4] + sq[5])
                        e = jnp.exp(-t)
                        a = e if a is None else a + e
                    accs.append(a)
                a = ((accs[0] + accs[1]) + (accs[2] + accs[3])) + accs[4]
                return (a_acc + a * m, a_cnt + m)

            return lax.fori_loop(0, nchunk, chunk, carry0)

        acc, cnt = lax.fori_loop(0, _BLK, row_loop, (acc, cnt))

    psum_st[...] = acc
    cnt_st[...] = cnt
    pltpu.sync_copy(psum_st, psum_h.at[pl.ds(wid * _L, _L)])
    pltpu.sync_copy(cnt_st, cnt_h.at[pl.ds(wid * _L, _L)])


def kernel(depth_pred, depth_gt, mask_gt, xy1_grid, hsv):
    B, _, H, W = depth_pred.shape
    r = _R
    padhw = ((0, 0), (r, r), (r, 2 + _WP - _W - 2 * r))
    pad3 = ((0, 0), (0, 0), (r, r), (r, 2 + _WP - _W - 2 * r))
    dpp = jnp.pad(depth_pred[:, 0], padhw, constant_values=_PAD)
    xy1p = jnp.pad(xy1_grid, pad3, constant_values=1.0)
    hsvp = jnp.pad(hsv, pad3, constant_values=_PAD)

    xy1f = xy1_grid.reshape(-1)
    dgtf = depth_gt.reshape(-1)
    mskf = mask_gt.astype(jnp.float32).reshape(-1)
    hsvf = hsv.reshape(-1)
    xy1pf = xy1p.reshape(-1)
    dppf = dpp.reshape(-1)
    hsvpf = hsvp.reshape(-1)

    mesh = plsc.VectorSubcoreMesh(core_axis_name="c", subcore_axis_name="s")
    fn = pl.kernel(
        _sc_body,
        out_type=[
            jax.ShapeDtypeStruct((_NW * _L,), jnp.float32),
            jax.ShapeDtypeStruct((_NW * _L,), jnp.float32),
        ],
        mesh=mesh,
        scratch_types=[
            pltpu.VMEM((3 * _PW * _WP,), jnp.float32),
            pltpu.VMEM((3 * _PW * _WP,), jnp.float32),
            pltpu.VMEM((_PW * _WP,), jnp.float32),
            pltpu.VMEM((3 * _BLK * _W,), jnp.float32),
            pltpu.VMEM((3 * _BLK * _W,), jnp.float32),
            pltpu.VMEM((_BLK * _W,), jnp.float32),
            pltpu.VMEM((_BLK * _W,), jnp.float32),
            pltpu.VMEM((_L,), jnp.float32),
            pltpu.VMEM((_L,), jnp.float32),
            pltpu.SemaphoreType.DMA,
        ],
    )
    psum, cnt = fn(xy1f, dgtf, mskf, hsvf, xy1pf, dppf, hsvpf)
    n_valid = jnp.sum(cnt)
    inp = jnp.sum(psum) / (n_valid * float((2 * r + 1) ** 2) + 1e-8)
    return 1.0 - inp


# speedup vs baseline: 3.9025x; 3.9025x over previous
import jax
import jax.numpy as jnp
from jax import lax
from jax.experimental import pallas as pl
from jax.experimental.pallas import tpu as pltpu
from jax.experimental.pallas import tpu_sc as plsc

_R = 2
_INV2SX = 1.0 / (2.0 * 0.05 ** 2)
_INV2SH = 1.0 / (2.0 * 0.1 ** 2)
_SX = _INV2SX ** 0.5
_SH = _INV2SH ** 0.5
_PAD = 1e4

_B, _H, _W = 4, 192, 640
_WP = 656
_HP = _H + 2 * _R
_NW = 32
_L = 16

_SC_B = 3
_BLK = _H // _NW
_PW = _BLK + 2 * _R


def _sc_body(xy1_h, dgt_h, msk_h, hsv_h, xy1p_h, dpp_h, hsvp_h,
             psum_h, cnt_h,
             xyzp_s, hsvp_s, dpp_v, xyzg_s, hsvg_s, dgt_v, msk_v,
             psum_st, cnt_st, sem):
    wid = lax.axis_index("s") * 2 + lax.axis_index("c")
    yg = wid * _BLK

    pb = (_SC_B * 3) * (_HP * _WP)
    gb = (_SC_B * 3) * (_H * _W)
    pb1 = _SC_B * (_HP * _WP)
    gb1 = _SC_B * (_H * _W)
    cps = []
    for c in range(3):
        cps.append(pltpu.async_copy(
            xy1p_h.at[pl.ds(pb + c * (_HP * _WP) + yg * _WP, _PW * _WP)],
            xyzp_s.at[pl.ds(c * (_PW * _WP), _PW * _WP)], sem))
    cps.append(pltpu.async_copy(
        dpp_h.at[pl.ds(pb1 + yg * _WP, _PW * _WP)], dpp_v, sem))
    for c in range(3):
        cps.append(pltpu.async_copy(
            hsvp_h.at[pl.ds(pb + c * (_HP * _WP) + yg * _WP, _PW * _WP)],
            hsvp_s.at[pl.ds(c * (_PW * _WP), _PW * _WP)], sem))
    for c in range(3):
        cps.append(pltpu.async_copy(
            xy1_h.at[pl.ds(gb + c * (_H * _W) + yg * _W, _BLK * _W)],
            xyzg_s.at[pl.ds(c * (_BLK * _W), _BLK * _W)], sem))
    cps.append(pltpu.async_copy(
        dgt_h.at[pl.ds(gb1 + yg * _W, _BLK * _W)], dgt_v, sem))
    for c in range(3):
        cps.append(pltpu.async_copy(
            hsv_h.at[pl.ds(gb + c * (_H * _W) + yg * _W, _BLK * _W)],
            hsvg_s.at[pl.ds(c * (_BLK * _W), _BLK * _W)], sem))
    cps.append(pltpu.async_copy(
        msk_h.at[pl.ds(gb1 + yg * _W, _BLK * _W)], msk_v, sem))
    for cp in cps:
        cp.wait()

    def pre_pred(j, carry):
        off = j * _L
        d = dpp_v[pl.ds(off, _L)] * _SX
        for c in range(3):
            sl = pl.ds(c * (_PW * _WP) + off, _L)
            xyzp_s[sl] = xyzp_s[sl] * d
            hsvp_s[sl] = hsvp_s[sl] * _SH
        return carry
    lax.fori_loop(0, _PW * _WP // _L, pre_pred, 0)

    def pre_gt(j, carry):
        off = j * _L
        d = dgt_v[pl.ds(off, _L)] * _SX
        for c in range(3):
            sl = pl.ds(c * (_BLK * _W) + off, _L)
            xyzg_s[sl] = xyzg_s[sl] * d
            hsvg_s[sl] = hsvg_s[sl] * _SH
        return carry
    lax.fori_loop(0, _BLK * _W // _L, pre_gt, 0)

    nchunk = _W // _L

    def chunk(j, carry):
        a_acc, a_cnt = carry
        ri = j // nchunk
        xc = j - ri * nchunk
        goff = j * _L
        gsl = pl.ds(goff, _L)
        m = msk_v[gsl]
        gx = [xyzg_s[pl.ds(c * (_BLK * _W) + goff, _L)] for c in range(3)]
        gh = [hsvg_s[pl.ds(c * (_BLK * _W) + goff, _L)] for c in range(3)]
        pbase = ri * _WP + xc * _L
        a = jnp.zeros((_L,), jnp.float32)
        for dy in range(2 * _R + 1):
            for dx in range(2 * _R + 1):
                nb = pbase + dy * _WP + dx
                t = jnp.zeros((_L,), jnp.float32)
                for c in range(3):
                    d = gx[c] - xyzp_s[pl.ds(c * (_PW * _WP) + nb, _L)]
                    t = t + d * d
                for c in range(3):
                    d = gh[c] - hsvp_s[pl.ds(c * (_PW * _WP) + nb, _L)]
                    t = t + d * d
                a = a + jnp.exp(-t)
        return (a_acc + a * m, a_cnt + m)

    acc, cnt = lax.fori_loop(
        0, _BLK * nchunk, chunk,
        (jnp.zeros((_L,), jnp.float32), jnp.zeros((_L,), jnp.float32)))

    psum_st[...] = acc
    cnt_st[...] = cnt
    pltpu.sync_copy(psum_st, psum_h.at[pl.ds(wid * _L, _L)])
    pltpu.sync_copy(cnt_st, cnt_h.at[pl.ds(wid * _L, _L)])


def _tc_body(xy1_ref, dgt_ref, msk_ref, hsv_ref,
             xy1p_ref, dpp_ref, hsvp_ref,
             psum_ref, cnt_ref):
    H, W = dgt_ref.shape[1], dgt_ref.shape[2]
    xy1 = xy1_ref[0]
    dgt = dgt_ref[0]
    hsv = hsv_ref[0]
    xyzg = xy1 * dgt[None]
    xy1p = xy1p_ref[0]
    dpp = dpp_ref[0]
    xyzp = xy1p * dpp[None]
    hsvp = hsvp_ref[0]
    total = jnp.zeros((H, W), dtype=jnp.float32)
    for dy in range(2 * _R + 1):
        for dx in range(2 * _R + 1):
            xs = xyzp[:, dy:dy + H, dx:dx + W]
            hs = hsvp[:, dy:dy + H, dx:dx + W]
            d2 = jnp.sum((xyzg - xs) ** 2, axis=0)
            h2 = jnp.sum((hsv - hs) ** 2, axis=0)
            total = total + jnp.exp(-(d2 * _INV2SX + h2 * _INV2SH))
    msk = msk_ref[0]
    psum_ref[0, 0, :] = jnp.full((128,), jnp.sum(total * msk), jnp.float32)
    cnt_ref[0, 0, :] = jnp.full((128,), jnp.sum(msk), jnp.float32)


def kernel(depth_pred, depth_gt, mask_gt, xy1_grid, hsv):
    B, _, H, W = depth_pred.shape
    r = _R
    padhw = ((0, 0), (r, r), (r, _WP - _W - r))
    pad3 = ((0, 0), (0, 0), (r, r), (r, _WP - _W - r))
    dgt = depth_gt[:, 0]
    msk = mask_gt[:, 0].astype(jnp.float32)
    dpp = jnp.pad(depth_pred[:, 0], padhw, constant_values=_PAD)
    xy1p = jnp.pad(xy1_grid, pad3, constant_values=1.0)
    hsvp = jnp.pad(hsv, pad3, constant_values=_PAD)

    mesh = plsc.VectorSubcoreMesh(core_axis_name="c", subcore_axis_name="s")
    sc_fn = pl.kernel(
        _sc_body,
        out_type=[
            jax.ShapeDtypeStruct((_NW * _L,), jnp.float32),
            jax.ShapeDtypeStruct((_NW * _L,), jnp.float32),
        ],
        mesh=mesh,
        scratch_types=[
            pltpu.VMEM((3 * _PW * _WP,), jnp.float32),
            pltpu.VMEM((3 * _PW * _WP,), jnp.float32),
            pltpu.VMEM((_PW * _WP,), jnp.float32),
            pltpu.VMEM((3 * _BLK * _W,), jnp.float32),
            pltpu.VMEM((3 * _BLK * _W,), jnp.float32),
            pltpu.VMEM((_BLK * _W,), jnp.float32),
            pltpu.VMEM((_BLK * _W,), jnp.float32),
            pltpu.VMEM((_L,), jnp.float32),
            pltpu.VMEM((_L,), jnp.float32),
            pltpu.SemaphoreType.DMA,
        ],
    )
    psum_sc, cnt_sc = sc_fn(
        xy1_grid.reshape(-1), dgt.reshape(-1), msk.reshape(-1),
        hsv.reshape(-1), xy1p.reshape(-1), dpp.reshape(-1),
        hsvp.reshape(-1))

    b3 = lambda b: (b, 0, 0, 0)
    b2 = lambda b: (b, 0, 0)
    psum_tc, cnt_tc = pl.pallas_call(
        _tc_body,
        grid=(_SC_B,),
        in_specs=[
            pl.BlockSpec((1, 3, H, W), b3),
            pl.BlockSpec((1, H, W), b2),
            pl.BlockSpec((1, H, W), b2),
            pl.BlockSpec((1, 3, H, W), b3),
            pl.BlockSpec((1, 3, _HP, _WP), b3),
            pl.BlockSpec((1, _HP, _WP), b2),
            pl.BlockSpec((1, 3, _HP, _WP), b3),
        ],
        out_specs=[
            pl.BlockSpec((1, 1, 128), lambda b: (b, 0, 0)),
            pl.BlockSpec((1, 1, 128), lambda b: (b, 0, 0)),
        ],
        out_shape=[
            jax.ShapeDtypeStruct((_SC_B, 1, 128), jnp.float32),
            jax.ShapeDtypeStruct((_SC_B, 1, 128), jnp.float32),
        ],
    )(xy1_grid, dgt, msk, hsv, xy1p, dpp, hsvp)

    psum = jnp.sum(psum_sc) + jnp.sum(psum_tc[:, 0, 0])
    n_valid = jnp.sum(cnt_sc) + jnp.sum(cnt_tc[:, 0, 0])
    inp = psum / (n_valid * float((2 * _R + 1) ** 2) + 1e-8)
    return 1.0 - inp


# speedup vs baseline: 4.7172x; 1.2088x over previous
import jax
import jax.numpy as jnp
from jax import lax
from jax.experimental import pallas as pl
from jax.experimental.pallas import tpu as pltpu
from jax.experimental.pallas import tpu_sc as plsc

_R = 2
_INV2SX = 1.0 / (2.0 * 0.05 ** 2)
_INV2SH = 1.0 / (2.0 * 0.1 ** 2)
_SX = _INV2SX ** 0.5
_SH = _INV2SH ** 0.5
_PAD = 1e4

_B, _H, _W = 4, 192, 640
_WP = 656
_HP = _H + 2 * _R
_NW = 32
_L = 16

_SC_B = 3
_BLK = _H // _NW
_PW = _BLK + 2 * _R


def _sc_body(xy1_h, dgt_h, msk_h, hsv_h, xy1p_h, dpp_h, hsvp_h,
             psum_h, cnt_h,
             xyzp_s, hsvp_s, dpp_v, xyzg_s, hsvg_s, dgt_v, msk_v,
             psum_st, cnt_st, sem):
    wid = lax.axis_index("s") * 2 + lax.axis_index("c")
    yg = wid * _BLK

    pb = 0
    gb = 0
    pb1 = 0
    gb1 = 0
    cps = []
    for c in range(3):
        cps.append(pltpu.async_copy(
            xy1p_h.at[pl.ds(pb + c * (_HP * _WP) + yg * _WP, _PW * _WP)],
            xyzp_s.at[pl.ds(c * (_PW * _WP), _PW * _WP)], sem))
    cps.append(pltpu.async_copy(
        dpp_h.at[pl.ds(pb1 + yg * _WP, _PW * _WP)], dpp_v, sem))
    for c in range(3):
        cps.append(pltpu.async_copy(
            hsvp_h.at[pl.ds(pb + c * (_HP * _WP) + yg * _WP, _PW * _WP)],
            hsvp_s.at[pl.ds(c * (_PW * _WP), _PW * _WP)], sem))
    for c in range(3):
        cps.append(pltpu.async_copy(
            xy1_h.at[pl.ds(gb + c * (_H * _W) + yg * _W, _BLK * _W)],
            xyzg_s.at[pl.ds(c * (_BLK * _W), _BLK * _W)], sem))
    cps.append(pltpu.async_copy(
        dgt_h.at[pl.ds(gb1 + yg * _W, _BLK * _W)], dgt_v, sem))
    for c in range(3):
        cps.append(pltpu.async_copy(
            hsv_h.at[pl.ds(gb + c * (_H * _W) + yg * _W, _BLK * _W)],
            hsvg_s.at[pl.ds(c * (_BLK * _W), _BLK * _W)], sem))
    cps.append(pltpu.async_copy(
        msk_h.at[pl.ds(gb1 + yg * _W, _BLK * _W)], msk_v, sem))
    for cp in cps:
        cp.wait()

    def pre_pred(j, carry):
        off = j * _L
        d = dpp_v[pl.ds(off, _L)] * _SX
        for c in range(3):
            sl = pl.ds(c * (_PW * _WP) + off, _L)
            xyzp_s[sl] = xyzp_s[sl] * d
            hsvp_s[sl] = hsvp_s[sl] * _SH
        return carry
    lax.fori_loop(0, _PW * _WP // _L, pre_pred, 0)

    def pre_gt(j, carry):
        off = j * _L
        d = dgt_v[pl.ds(off, _L)] * _SX
        for c in range(3):
            sl = pl.ds(c * (_BLK * _W) + off, _L)
            xyzg_s[sl] = xyzg_s[sl] * d
            hsvg_s[sl] = hsvg_s[sl] * _SH
        return carry
    lax.fori_loop(0, _BLK * _W // _L, pre_gt, 0)

    nchunk = _W // _L

    def chunk(j, carry):
        a_acc, a_cnt = carry
        ri = j // nchunk
        xc = j - ri * nchunk
        goff = j * _L
        gsl = pl.ds(goff, _L)
        m = msk_v[gsl]
        gx = [xyzg_s[pl.ds(c * (_BLK * _W) + goff, _L)] for c in range(3)]
        gh = [hsvg_s[pl.ds(c * (_BLK * _W) + goff, _L)] for c in range(3)]
        pbase = ri * _WP + xc * _L
        a = jnp.zeros((_L,), jnp.float32)
        for dy in range(2 * _R + 1):
            for dx in range(2 * _R + 1):
                nb = pbase + dy * _WP + dx
                t = jnp.zeros((_L,), jnp.float32)
                for c in range(3):
                    d = gx[c] - xyzp_s[pl.ds(c * (_PW * _WP) + nb, _L)]
                    t = t + d * d
                for c in range(3):
                    d = gh[c] - hsvp_s[pl.ds(c * (_PW * _WP) + nb, _L)]
                    t = t + d * d
                a = a + jnp.exp(-t)
        return (a_acc + a * m, a_cnt + m)

    acc, cnt = lax.fori_loop(
        0, _BLK * nchunk, chunk,
        (jnp.zeros((_L,), jnp.float32), jnp.zeros((_L,), jnp.float32)))

    psum_st[...] = acc
    cnt_st[...] = cnt
    pltpu.sync_copy(psum_st, psum_h.at[pl.ds(wid * _L, _L)])
    pltpu.sync_copy(cnt_st, cnt_h.at[pl.ds(wid * _L, _L)])


def _tc_body(xy1_ref, dgt_ref, msk_ref, hsv_ref,
             xy1p_ref, dpp_ref, hsvp_ref,
             psum_ref, cnt_ref):
    H, W = dgt_ref.shape[1], dgt_ref.shape[2]
    xy1 = xy1_ref[0]
    dgt = dgt_ref[0]
    hsv = hsv_ref[0]
    xyzg = xy1 * dgt[None]
    xy1p = xy1p_ref[0]
    dpp = dpp_ref[0]
    xyzp = xy1p * dpp[None]
    hsvp = hsvp_ref[0]
    total = jnp.zeros((H, W), dtype=jnp.float32)
    for dy in range(2 * _R + 1):
        for dx in range(2 * _R + 1):
            xs = xyzp[:, dy:dy + H, dx:dx + W]
            hs = hsvp[:, dy:dy + H, dx:dx + W]
            d2 = jnp.sum((xyzg - xs) ** 2, axis=0)
            h2 = jnp.sum((hsv - hs) ** 2, axis=0)
            total = total + jnp.exp(-(d2 * _INV2SX + h2 * _INV2SH))
    msk = msk_ref[0]
    psum_ref[0, 0, :] = jnp.full((128,), jnp.sum(total * msk), jnp.float32)
    cnt_ref[0, 0, :] = jnp.full((128,), jnp.sum(msk), jnp.float32)


def kernel(depth_pred, depth_gt, mask_gt, xy1_grid, hsv):
    B, _, H, W = depth_pred.shape
    r = _R
    padhw = ((0, 0), (r, r), (r, _WP - _W - r))
    pad3 = ((0, 0), (0, 0), (r, r), (r, _WP - _W - r))
    dgt = depth_gt[:, 0]
    msk = mask_gt[:, 0].astype(jnp.float32)
    dpp = jnp.pad(depth_pred[:, 0], padhw, constant_values=_PAD)
    xy1p = jnp.pad(xy1_grid, pad3, constant_values=1.0)
    hsvp = jnp.pad(hsv, pad3, constant_values=_PAD)

    mesh = plsc.VectorSubcoreMesh(core_axis_name="c", subcore_axis_name="s")
    sc_fn = pl.kernel(
        _sc_body,
        out_type=[
            jax.ShapeDtypeStruct((_NW * _L,), jnp.float32),
            jax.ShapeDtypeStruct((_NW * _L,), jnp.float32),
        ],
        mesh=mesh,
        scratch_types=[
            pltpu.VMEM((3 * _PW * _WP,), jnp.float32),
            pltpu.VMEM((3 * _PW * _WP,), jnp.float32),
            pltpu.VMEM((_PW * _WP,), jnp.float32),
            pltpu.VMEM((3 * _BLK * _W,), jnp.float32),
            pltpu.VMEM((3 * _BLK * _W,), jnp.float32),
            pltpu.VMEM((_BLK * _W,), jnp.float32),
            pltpu.VMEM((_BLK * _W,), jnp.float32),
            pltpu.VMEM((_L,), jnp.float32),
            pltpu.VMEM((_L,), jnp.float32),
            pltpu.SemaphoreType.DMA,
        ],
    )
    psum_sc, cnt_sc = sc_fn(
        xy1_grid[_SC_B].reshape(-1), dgt[_SC_B].reshape(-1),
        msk[_SC_B].reshape(-1), hsv[_SC_B].reshape(-1),
        xy1p[_SC_B].reshape(-1), dpp[_SC_B].reshape(-1),
        hsvp[_SC_B].reshape(-1))

    b3 = lambda b: (b, 0, 0, 0)
    b2 = lambda b: (b, 0, 0)
    psum_tc, cnt_tc = pl.pallas_call(
        _tc_body,
        grid=(_SC_B,),
        in_specs=[
            pl.BlockSpec((1, 3, H, W), b3),
            pl.BlockSpec((1, H, W), b2),
            pl.BlockSpec((1, H, W), b2),
            pl.BlockSpec((1, 3, H, W), b3),
            pl.BlockSpec((1, 3, _HP, _WP), b3),
            pl.BlockSpec((1, _HP, _WP), b2),
            pl.BlockSpec((1, 3, _HP, _WP), b3),
        ],
        out_specs=[
            pl.BlockSpec((1, 1, 128), lambda b: (b, 0, 0)),
            pl.BlockSpec((1, 1, 128), lambda b: (b, 0, 0)),
        ],
        out_shape=[
            jax.ShapeDtypeStruct((_SC_B, 1, 128), jnp.float32),
            jax.ShapeDtypeStruct((_SC_B, 1, 128), jnp.float32),
        ],
    )(xy1_grid, dgt, msk, hsv, xy1p, dpp, hsvp)

    psum = jnp.sum(psum_sc) + jnp.sum(psum_tc[:, 0, 0])
    n_valid = jnp.sum(cnt_sc) + jnp.sum(cnt_tc[:, 0, 0])
    inp = psum / (n_valid * float((2 * _R + 1) ** 2) + 1e-8)
    return 1.0 - inp
